# Initial kernel scaffold; baseline (speedup 1.0000x reference)
import jax
import jax.numpy as jnp
from jax.experimental import pallas as pl


def kernel(token_centers, bias_table):
    raise NotImplementedError("write your pallas kernel here")


# speedup vs baseline: 16.2923x; 16.2923x over previous
import functools

import jax
import jax.numpy as jnp
from jax import lax
from jax.experimental import pallas as pl
from jax.experimental.pallas import tpu as pltpu
from jax.experimental.pallas import tpu_sc as plsc

NUM_BINS = 8
INV_BIN = 8.0
NUM_HEADS = 16
SIDE = 2 * NUM_BINS + 1
TABLE_SIZE = SIDE ** 3

NC, NS, L = 2, 16, 16
NW = NC * NS


def _splat(x):
    return jnp.broadcast_to(jnp.asarray(x, jnp.int32), (L,))


def _body(coords_hbm, table_hbm, out_hbm, coords_v, table_v, out_v,
          sem0, sem1, B, N):
    rows_per_w = (B * N) // NW
    wid = lax.axis_index("s") * NC + lax.axis_index("c")

    pltpu.sync_copy(table_hbm, table_v)
    pltpu.sync_copy(coords_hbm, coords_v)

    iota = lax.iota(jnp.int32, L)
    sems = (sem0, sem1)
    n_chunks = N // L

    def row_pair(t2, _):
        for k in range(2):
            tg = wid * rows_per_w + t2 * 2 + k
            b = tg // N
            i = tg % N
            sb = jnp.broadcast_to(b, (L,))
            si = jnp.broadcast_to(i, (L,))
            xi = plsc.load_gather(coords_v, [sb * 3 + 0, si])
            yi = plsc.load_gather(coords_v, [sb * 3 + 1, si])
            zi = plsc.load_gather(coords_v, [sb * 3 + 2, si])

            @pl.when(t2 > 0)
            def _():
                pltpu.make_async_copy(
                    out_v.at[k], out_hbm.at[0, :, 0, :], sems[k]).wait()

            def chunk(jc, _):
                for u in range(4):
                    jb = (jc * 4 + u) * L
                    jv = iota + jb
                    xj = plsc.load_gather(coords_v, [sb * 3 + 0, jv])
                    yj = plsc.load_gather(coords_v, [sb * 3 + 1, jv])
                    zj = plsc.load_gather(coords_v, [sb * 3 + 2, jv])
                    qx = jnp.clip((xi - xj) * INV_BIN, -8.49, 8.49) + 8.5
                    qy = jnp.clip((yi - yj) * INV_BIN, -8.49, 8.49) + 8.5
                    qz = jnp.clip((zi - zj) * INV_BIN, -8.49, 8.49) + 8.5
                    idx = (qx.astype(jnp.int32) * (SIDE * SIDE)
                           + qy.astype(jnp.int32) * SIDE
                           + qz.astype(jnp.int32))
                    for h in range(NUM_HEADS):
                        val = plsc.load_gather(table_v, [_splat(h), idx])
                        out_v[k, h, pl.ds(jb, L)] = val
                return 0

            lax.fori_loop(0, n_chunks // 4, chunk, 0)
            pltpu.async_copy(out_v.at[k], out_hbm.at[b, :, i, :], sems[k])
        return 0

    lax.fori_loop(0, rows_per_w // 2, row_pair, 0)
    for k in range(2):
        pltpu.make_async_copy(
            out_v.at[k], out_hbm.at[0, :, 0, :], sems[k]).wait()


def kernel(token_centers, bias_table):
    B, N, _ = token_centers.shape
    H = bias_table.shape[0]
    assert H == NUM_HEADS and bias_table.shape[1] == TABLE_SIZE
    assert (B * N) % (2 * NW) == 0 and N % (4 * L) == 0

    coords = jnp.transpose(token_centers, (0, 2, 1)).reshape(B * 3, N)

    mesh = plsc.VectorSubcoreMesh(
        core_axis_name="c", subcore_axis_name="s",
        num_cores=NC, num_subcores=NS)
    body = functools.partial(_body, B=B, N=N)
    f = pl.kernel(
        body,
        out_type=jax.ShapeDtypeStruct((B, H, N, N), jnp.float32),
        mesh=mesh,
        compiler_params=pltpu.CompilerParams(
            use_tc_tiling_on_sc=False, needs_layout_passes=False),
        scratch_types=[
            pltpu.VMEM((B * 3, N), jnp.float32),
            pltpu.VMEM((H, TABLE_SIZE), jnp.float32),
            pltpu.VMEM((2, H, N), jnp.float32),
            pltpu.SemaphoreType.DMA,
            pltpu.SemaphoreType.DMA,
        ],
    )
    return f(coords, bias_table)


# speedup vs baseline: 35.4052x; 2.1731x over previous
import functools

import jax
import jax.numpy as jnp
from jax import lax
from jax.experimental import pallas as pl
from jax.experimental.pallas import tpu as pltpu
from jax.experimental.pallas import tpu_sc as plsc

NUM_BINS = 8
INV_BIN = 8.0
NUM_HEADS = 16
SIDE = 2 * NUM_BINS + 1
TABLE_SIZE = SIDE ** 3

NC, NS, L = 2, 16, 16
NW = NC * NS


def _splat(x):
    return jnp.broadcast_to(jnp.asarray(x, jnp.int32), (L,))


def _body(coords_hbm, table_hbm, out_hbm, coords_v, table_v, out_v,
          sem0, sem1, B, N):
    rows_per_w = (B * N) // NW
    wid = lax.axis_index("s") * NC + lax.axis_index("c")

    pltpu.sync_copy(table_hbm, table_v)
    pltpu.sync_copy(coords_hbm, coords_v)

    iota = lax.iota(jnp.int32, L)
    sems = (sem0, sem1)
    n_chunks = N // L

    def row_pair(t2, _):
        for k in range(2):
            tg = wid * rows_per_w + t2 * 2 + k
            b = tg // N
            i = tg % N
            sb = jnp.broadcast_to(b, (L,))
            si = jnp.broadcast_to(i, (L,))
            xi = plsc.load_gather(coords_v, [sb * 3 + 0, si])
            yi = plsc.load_gather(coords_v, [sb * 3 + 1, si])
            zi = plsc.load_gather(coords_v, [sb * 3 + 2, si])

            @pl.when(t2 > 0)
            def _():
                pltpu.make_async_copy(
                    out_v.at[k], out_hbm.at[0, :, 0, :], sems[k]).wait()

            @plsc.parallel_loop(0, n_chunks, unroll=4)
            def chunk(jc):
                jb = jc * L
                jv = iota + jb
                xj = plsc.load_gather(coords_v, [sb * 3 + 0, jv])
                yj = plsc.load_gather(coords_v, [sb * 3 + 1, jv])
                zj = plsc.load_gather(coords_v, [sb * 3 + 2, jv])
                qx = jnp.clip((xi - xj) * INV_BIN, -8.49, 8.49) + 8.5
                qy = jnp.clip((yi - yj) * INV_BIN, -8.49, 8.49) + 8.5
                qz = jnp.clip((zi - zj) * INV_BIN, -8.49, 8.49) + 8.5
                idx = (qx.astype(jnp.int32) * (SIDE * SIDE)
                       + qy.astype(jnp.int32) * SIDE
                       + qz.astype(jnp.int32))
                for h in range(NUM_HEADS):
                    val = plsc.load_gather(table_v, [_splat(h), idx])
                    out_v[k, h, pl.ds(jb, L)] = val
            pltpu.async_copy(out_v.at[k], out_hbm.at[b, :, i, :], sems[k])
        return 0

    lax.fori_loop(0, rows_per_w // 2, row_pair, 0)
    for k in range(2):
        pltpu.make_async_copy(
            out_v.at[k], out_hbm.at[0, :, 0, :], sems[k]).wait()


def kernel(token_centers, bias_table):
    B, N, _ = token_centers.shape
    H = bias_table.shape[0]
    assert H == NUM_HEADS and bias_table.shape[1] == TABLE_SIZE
    assert (B * N) % (2 * NW) == 0 and N % (4 * L) == 0

    coords = jnp.transpose(token_centers, (0, 2, 1)).reshape(B * 3, N)

    mesh = plsc.VectorSubcoreMesh(
        core_axis_name="c", subcore_axis_name="s",
        num_cores=NC, num_subcores=NS)
    body = functools.partial(_body, B=B, N=N)
    f = pl.kernel(
        body,
        out_type=jax.ShapeDtypeStruct((B, H, N, N), jnp.float32),
        mesh=mesh,
        compiler_params=pltpu.CompilerParams(
            use_tc_tiling_on_sc=False, needs_layout_passes=False),
        scratch_types=[
            pltpu.VMEM((B * 3, N), jnp.float32),
            pltpu.VMEM((H, TABLE_SIZE), jnp.float32),
            pltpu.VMEM((2, H, N), jnp.float32),
            pltpu.SemaphoreType.DMA,
            pltpu.SemaphoreType.DMA,
        ],
    )
    return f(coords, bias_table)


# speedup vs baseline: 77.4094x; 2.1864x over previous
import functools

import jax
import jax.numpy as jnp
from jax import lax
from jax.experimental import pallas as pl
from jax.experimental.pallas import tpu as pltpu
from jax.experimental.pallas import tpu_sc as plsc

NUM_BINS = 8
INV_BIN = 8.0
NUM_HEADS = 16
SIDE = 2 * NUM_BINS + 1
TABLE_SIZE = SIDE ** 3

NC, NS, L = 2, 16, 16
NW = NC * NS


def _splat(x):
    return jnp.broadcast_to(jnp.asarray(x, jnp.int32), (L,))


def _body(coords_hbm, table_hbm, out_hbm, coords_v, table_v, out_v,
          sem0, sem1, B, N):
    rows_per_w = (B * N) // NW
    wid = lax.axis_index("s") * NC + lax.axis_index("c")

    pltpu.sync_copy(table_hbm, table_v)
    pltpu.sync_copy(coords_hbm, coords_v)

    iota = lax.iota(jnp.int32, L)
    sems = (sem0, sem1)
    n_chunks = N // L

    def row_pair(t2, _):
        for k in range(2):
            tg = wid * rows_per_w + t2 * 2 + k
            b = tg // N
            i = tg % N
            sb = jnp.broadcast_to(b, (L,))
            si = jnp.broadcast_to(i, (L,))
            xi = plsc.load_gather(coords_v, [sb * 3 + 0, si])
            yi = plsc.load_gather(coords_v, [sb * 3 + 1, si])
            zi = plsc.load_gather(coords_v, [sb * 3 + 2, si])

            @pl.when(t2 > 0)
            def _():
                pltpu.make_async_copy(
                    out_v.at[k], out_hbm.at[0, :, 0, :], sems[k]).wait()

            @plsc.parallel_loop(0, n_chunks, unroll=4)
            def chunk(jc):
                jb = jc * L
                jv = iota + jb
                xj = plsc.load_gather(coords_v, [sb * 3 + 0, jv])
                yj = plsc.load_gather(coords_v, [sb * 3 + 1, jv])
                zj = plsc.load_gather(coords_v, [sb * 3 + 2, jv])
                qx = jnp.clip((xi - xj) * INV_BIN, -8.49, 8.49) + 8.5
                qy = jnp.clip((yi - yj) * INV_BIN, -8.49, 8.49) + 8.5
                qz = jnp.clip((zi - zj) * INV_BIN, -8.49, 8.49) + 8.5
                idx = (qx.astype(jnp.int32) * (SIDE * SIDE)
                       + qy.astype(jnp.int32) * SIDE
                       + qz.astype(jnp.int32))
                for h in range(NUM_HEADS):
                    val = plsc.load_gather(table_v, [_splat(h), idx])
                    out_v[k, h, pl.ds(jb, L)] = val
            pltpu.async_copy(out_v.at[k], out_hbm.at[b, :, i, :], sems[k])
        return 0

    lax.fori_loop(0, rows_per_w // 2, row_pair, 0)
    for k in range(2):
        pltpu.make_async_copy(
            out_v.at[k], out_hbm.at[0, :, 0, :], sems[k]).wait()


def kernel(token_centers, bias_table):
    B, N, _ = token_centers.shape
    H = bias_table.shape[0]
    assert H == NUM_HEADS and bias_table.shape[1] == TABLE_SIZE
    assert (B * N) % (2 * NW) == 0 and N % (4 * L) == 0

    coords = jnp.transpose(token_centers, (0, 2, 1)).reshape(B * 3, N)

    mesh = plsc.VectorSubcoreMesh(
        core_axis_name="c", subcore_axis_name="s",
        num_cores=NC, num_subcores=NS)
    body = functools.partial(_body, B=B, N=N)
    f = pl.kernel(
        body,
        out_type=jax.ShapeDtypeStruct((B, H, N, N), jnp.float32),
        mesh=mesh,
        compiler_params=pltpu.CompilerParams(
            needs_layout_passes=False),
        scratch_types=[
            pltpu.VMEM((B * 3, N), jnp.float32),
            pltpu.VMEM((H, TABLE_SIZE), jnp.float32),
            pltpu.VMEM((2, H, N), jnp.float32),
            pltpu.SemaphoreType.DMA,
            pltpu.SemaphoreType.DMA,
        ],
    )
    return f(coords, bias_table)
